# speedup vs baseline: 4.4988x; 4.4988x over previous
import jax
import jax.numpy as jnp
from jax.experimental import pallas as pl

_K = 32


def _head_kernel(q_ref, k_ref, p_ref, idx_ref):
    q = q_ref[0]
    k = k_ref[0]
    s = jax.lax.dot_general(q, k, (((1,), (1,)), ((), ())),
                            preferred_element_type=jnp.float32)
    sq, n_kv = s.shape
    lane = jax.lax.broadcasted_iota(jnp.int32, s.shape, 1)
    rank_lane = jax.lax.broadcasted_iota(jnp.int32, (sq, _K), 1)
    neg = jnp.float32(-jnp.inf)

    def body(t, carry):
        s, topv, topi = carry
        mu = jnp.max(s, axis=1, keepdims=True)
        eq = s == mu
        idx = jnp.min(jnp.where(eq, lane, n_kv), axis=1, keepdims=True)
        kill = eq & (lane == idx)
        s = jnp.where(kill, neg, s)
        topv = jnp.where(rank_lane == t, mu, topv)
        topi = jnp.where(rank_lane == t, idx, topi)
        return s, topv, topi

    topv0 = jnp.full((sq, _K), neg, jnp.float32)
    topi0 = jnp.zeros((sq, _K), jnp.int32)
    _, topv, topi = jax.lax.fori_loop(0, _K, body, (s, topv0, topi0))

    e = jnp.exp(topv - topv[:, 0:1])
    p = e / jnp.sum(e, axis=1, keepdims=True)
    p_ref[0] = p
    idx_ref[0] = topi


def _select_probs(Qf, Kf):
    G, Sq, D = Qf.shape
    Skv = Kf.shape[1]
    return pl.pallas_call(
        _head_kernel,
        grid=(G,),
        in_specs=[pl.BlockSpec((1, Sq, D), lambda g: (g, 0, 0)),
                  pl.BlockSpec((1, Skv, D), lambda g: (g, 0, 0))],
        out_specs=[pl.BlockSpec((1, Sq, _K), lambda g: (g, 0, 0)),
                   pl.BlockSpec((1, Sq, _K), lambda g: (g, 0, 0))],
        out_shape=[jax.ShapeDtypeStruct((G, Sq, _K), jnp.float32),
                   jax.ShapeDtypeStruct((G, Sq, _K), jnp.int32)],
    )(Qf, Kf)


def kernel(Q, K, V):
    B, H, Sq, D = Q.shape
    Skv = K.shape[2]
    G = B * H
    Qf = Q.reshape(G, Sq, D)
    Kf = K.reshape(G, Skv, D)
    p, idx = _select_probs(Qf, Kf)
    Vf = V[..., :_K].reshape(G, Skv, _K)
    tv = jnp.take_along_axis(Vf, idx, axis=1)
    out = jnp.matmul(p, tv)
    return out.reshape(B, H, Sq, _K)
